# Initial kernel scaffold; baseline (speedup 1.0000x reference)
import jax
import jax.numpy as jnp
from jax.experimental import pallas as pl


def kernel(x, edge_index, w1l, b1, w1r, w2l, b2, w2r, w3l, b3, w3r, wc, bc):
    raise NotImplementedError("write your pallas kernel here")


# speedup vs baseline: 26.4627x; 26.4627x over previous
import functools

import jax
import jax.numpy as jnp
from jax import lax
from jax.experimental import pallas as pl
from jax.experimental.pallas import tpu as pltpu
from jax.experimental.pallas import tpu_sc as plsc

NC = 2
NS = 16
NW = NC * NS

K = 2048


def _pad_edges(edge_index, n):
    e = edge_index.shape[1]
    unit = K * NW
    e_pad = ((e + unit - 1) // unit) * unit
    pad = e_pad - e
    src = edge_index[0]
    dst = edge_index[1]
    if pad:
        src = jnp.concatenate([src, jnp.zeros((pad,), jnp.int32)])
        dst = jnp.concatenate([dst, jnp.full((pad,), n, jnp.int32)])
    return src, dst


@functools.partial(jax.jit, static_argnums=(3, 4))
def _sc_edge_pass(cols, src1, dst1, n_pad, count_deg):
    d = len(cols)
    nacc = d + (1 if count_deg else 0)
    e_pad = src1.shape[0]
    cpw = e_pad // K // NW
    tr = n_pad // NS

    def body(*refs):
        tables = refs[:d]
        src_h, dst_h, z_h = refs[d:d + 3]
        outs = refs[d + 3:d + 3 + nacc]
        accs = refs[d + 3 + nacc:d + 3 + 2 * nacc]
        sidx, didx = refs[d + 3 + 2 * nacc:d + 5 + 2 * nacc]
        msgs = refs[d + 5 + 2 * nacc:d + 5 + 2 * nacc + d]
        ones_v = refs[-3]
        gsem, ssem = refs[-2], refs[-1]

        c = lax.axis_index("c")
        s = lax.axis_index("s")
        wid = s * NC + c

        if count_deg:
            one = jnp.full((16,), 1.0, jnp.float32)
            def fill(i, carry):
                ones_v[pl.ds(i * 16, 16)] = one
                return carry
            lax.fori_loop(0, K // 16, fill, 0)

        for a in accs:
            pltpu.sync_copy(z_h.at[pl.ds(s * tr, tr)], a.at[pl.ds(s * tr, tr)])
        plsc.subcore_barrier()

        def chunk(o, carry):
            base = (wid * cpw + o) * K
            pltpu.sync_copy(src_h.at[pl.ds(base, K)], sidx)
            pltpu.sync_copy(dst_h.at[pl.ds(base, K)], didx)
            hs = [pltpu.async_copy(tables[j].at[sidx], msgs[j], gsem)
                  for j in range(d)]
            for h in hs:
                h.wait()
            hs = [pltpu.async_copy(msgs[j], accs[j].at[didx], ssem, add=True)
                  for j in range(d)]
            if count_deg:
                hs.append(pltpu.async_copy(ones_v, accs[d].at[didx], ssem,
                                           add=True))
            for h in hs:
                h.wait()
            return carry

        lax.fori_loop(0, cpw, chunk, 0)
        plsc.subcore_barrier()

        for a, o in zip(accs, outs):
            pltpu.sync_copy(a.at[pl.ds(s * tr, tr)],
                            o.at[pl.ds(c * n_pad + s * tr, tr)])

    z = jnp.zeros((n_pad,), jnp.float32)
    mesh = plsc.VectorSubcoreMesh(core_axis_name="c", subcore_axis_name="s")
    return pl.kernel(
        body,
        out_type=[jax.ShapeDtypeStruct((NC * n_pad,), jnp.float32)] * nacc,
        mesh=mesh,
        scratch_types=(
            [pltpu.VMEM_SHARED((n_pad,), jnp.float32)] * nacc
            + [pltpu.VMEM((K,), jnp.int32)] * 2
            + [pltpu.VMEM((K,), jnp.float32)] * d
            + [pltpu.VMEM((K,), jnp.float32),
               pltpu.SemaphoreType.DMA,
               pltpu.SemaphoreType.DMA]
        ),
        compiler_params=pltpu.CompilerParams(use_tc_tiling_on_sc=False),
    )(*cols, src1, dst1, z)


_BN = 1000


def _mm(a, w, kdim):
    a = a.astype(jnp.bfloat16).astype(jnp.float32)
    w = w.astype(jnp.bfloat16).astype(jnp.float32)
    return sum(a[:, k:k + 1] * w[k:k + 1, :] for k in range(kdim))


def _node_spec(d):
    return pl.BlockSpec((_BN, d), lambda i: (i, 0))


def _w_spec(shape):
    return pl.BlockSpec(shape, lambda i: (0, 0))


def _dense1(ps, pd, x, w1l, b1, w1r):
    n = x.shape[0]

    def body(ps0, ps1, pd0, pd1, x_r, wl_r, b_r, wr_r, h1_r, inv_r):
        inv = 1.0 / jnp.maximum(pd0[...] + pd1[...], 1.0)
        mean = (ps0[...] + ps1[...]) * inv
        h1 = mean * wl_r[...] + b_r[...] + x_r[...] * wr_r[...]
        h1_r[...] = jnp.maximum(h1, 0.0)
        inv_r[...] = inv

    return pl.pallas_call(
        body,
        grid=(n // _BN,),
        in_specs=[_node_spec(1)] * 5 + [_w_spec((1, 4))] * 3,
        out_specs=[_node_spec(4), _node_spec(1)],
        out_shape=[jax.ShapeDtypeStruct((n, 4), jnp.float32),
                   jax.ShapeDtypeStruct((n, 1), jnp.float32)],
    )(ps[0], ps[1], pd[0], pd[1], x, w1l, b1, w1r)


def _dense2(pcols, inv, h1, w2l, b2, w2r):
    n = h1.shape[0]

    def body(p00, p01, p10, p11, p20, p21, p30, p31, inv_r, h1_r,
             wl_r, b_r, wr_r, h2_r):
        iv = inv_r[...]
        mean = jnp.concatenate(
            [(p00[...] + p01[...]) * iv, (p10[...] + p11[...]) * iv,
             (p20[...] + p21[...]) * iv, (p30[...] + p31[...]) * iv], axis=1)
        h2 = _mm(mean, wl_r[...], 4) + b_r[...] + _mm(h1_r[...], wr_r[...], 4)
        h2_r[...] = jnp.maximum(h2, 0.0)

    return pl.pallas_call(
        body,
        grid=(n // _BN,),
        in_specs=([_node_spec(1)] * 9 + [_node_spec(4)]
                  + [_w_spec((4, 4)), _w_spec((1, 4)), _w_spec((4, 4))]),
        out_specs=[_node_spec(4)],
        out_shape=[jax.ShapeDtypeStruct((n, 4), jnp.float32)],
    )(*pcols, inv, h1, w2l, b2, w2r)[0]


def _dense3(pcols, inv, h2, w3l, b3, w3r, wc, bc):
    n = h2.shape[0]

    def body(p00, p01, p10, p11, p20, p21, p30, p31, inv_r, h2_r,
             wl_r, b_r, wr_r, wc_r, bc_r, out_r, h3_r):
        iv = inv_r[...]
        mean = jnp.concatenate(
            [(p00[...] + p01[...]) * iv, (p10[...] + p11[...]) * iv,
             (p20[...] + p21[...]) * iv, (p30[...] + p31[...]) * iv], axis=1)
        h3 = _mm(mean, wl_r[...], 4) + b_r[...] + _mm(h2_r[...], wr_r[...], 4)
        h3 = jnp.maximum(h3, 0.0)
        h3_r[...] = h3
        out_r[...] = _mm(h3, wc_r[...], 2) + bc_r[...]

    return pl.pallas_call(
        body,
        grid=(n // _BN,),
        in_specs=([_node_spec(1)] * 9 + [_node_spec(4)]
                  + [_w_spec((4, 2)), _w_spec((1, 2)), _w_spec((4, 2)),
                     _w_spec((2, 2)), _w_spec((1, 2))]),
        out_specs=[_node_spec(2), _node_spec(2)],
        out_shape=[jax.ShapeDtypeStruct((n, 2), jnp.float32),
                   jax.ShapeDtypeStruct((n, 2), jnp.float32)],
    )(*pcols, inv, h2, w3l, b3, w3r, wc, bc)


def kernel(x, edge_index, w1l, b1, w1r, w2l, b2, w2r, w3l, b3, w3r, wc, bc):
    n = x.shape[0]
    n_pad = ((n + 1 + 8 * NS - 1) // (8 * NS)) * (8 * NS)

    src1, dst1 = _pad_edges(edge_index, n)

    def split(raw):
        return raw[:n].reshape(n, 1), raw[n_pad:n_pad + n].reshape(n, 1)

    outs = _sc_edge_pass((x[:, 0],), src1, dst1, n_pad, True)
    h1, inv = _dense1(split(outs[0]), split(outs[1]), x,
                      w1l, b1.reshape(1, 4), w1r)

    outs = _sc_edge_pass(tuple(h1[:, j] for j in range(4)), src1, dst1,
                         n_pad, False)
    pcols = [p for o in outs for p in split(o)]
    h2 = _dense2(pcols, inv, h1, w2l, b2.reshape(1, 4), w2r)

    outs = _sc_edge_pass(tuple(h2[:, j] for j in range(4)), src1, dst1,
                         n_pad, False)
    pcols = [p for o in outs for p in split(o)]
    out, h3 = _dense3(pcols, inv, h2, w3l, b3.reshape(1, 2), w3r, wc,
                      bc.reshape(1, 2))
    return (out, h3)


# speedup vs baseline: 36.4598x; 1.3778x over previous
import functools

import jax
import jax.numpy as jnp
from jax import lax
from jax.experimental import pallas as pl
from jax.experimental.pallas import tpu as pltpu
from jax.experimental.pallas import tpu_sc as plsc

NC = 2
NS = 16
NW = NC * NS

K = 4000


def _pad_edges(edge_index, n):
    e = edge_index.shape[1]
    unit = K * NW
    e_pad = ((e + unit - 1) // unit) * unit
    pad = e_pad - e
    src = edge_index[0]
    dst = edge_index[1]
    if pad:
        src = jnp.concatenate([src, jnp.zeros((pad,), jnp.int32)])
        dst = jnp.concatenate([dst, jnp.full((pad,), n, jnp.int32)])
    return src, dst


@functools.partial(jax.jit, static_argnums=(3, 4))
def _sc_edge_pass(cols, src1, dst1, n_pad, count_deg):
    d = len(cols)
    nacc = d + (1 if count_deg else 0)
    e_pad = src1.shape[0]
    cpw = e_pad // K // NW
    tr = n_pad // NS

    assert cpw % 2 == 0

    def body(*refs):
        tables = refs[:d]
        src_h, dst_h, z_h = refs[d:d + 3]
        outs = refs[d + 3:d + 3 + nacc]
        accs = refs[d + 3 + nacc:d + 3 + 2 * nacc]
        sidxs = refs[d + 3 + 2 * nacc:d + 5 + 2 * nacc]
        didxs = refs[d + 5 + 2 * nacc:d + 7 + 2 * nacc]
        msgs = [refs[d + 7 + 2 * nacc + p * d:d + 7 + 2 * nacc + (p + 1) * d]
                for p in range(2)]
        ones_v = refs[-3]
        gsem, ssem = refs[-2], refs[-1]

        c = lax.axis_index("c")
        s = lax.axis_index("s")
        wid = s * NC + c

        if count_deg:
            one = jnp.full((16,), 1.0, jnp.float32)
            def fill(i, carry):
                ones_v[pl.ds(i * 16, 16)] = one
                return carry
            lax.fori_loop(0, K // 16, fill, 0)

        for a in accs:
            pltpu.sync_copy(z_h.at[pl.ds(s * tr, tr)], a.at[pl.ds(s * tr, tr)])
        plsc.subcore_barrier()

        def stage(o, p):
            base = (wid * cpw + o) * K
            pltpu.sync_copy(src_h.at[pl.ds(base, K)], sidxs[p])
            pltpu.sync_copy(dst_h.at[pl.ds(base, K)], didxs[p])

        def g_descs(p):
            return [pltpu.make_async_copy(tables[j].at[sidxs[p]],
                                          msgs[p][j], gsem)
                    for j in range(d)]

        def s_descs(p):
            hs = [pltpu.make_async_copy(msgs[p][j], accs[j].at[didxs[p]],
                                        ssem)
                  for j in range(d)]
            if count_deg:
                hs.append(pltpu.make_async_copy(ones_v, accs[d].at[didxs[p]],
                                                ssem))
            return hs

        def fire_g(p):
            for j in range(d):
                pltpu.async_copy(tables[j].at[sidxs[p]], msgs[p][j], gsem)

        def fire_s(p):
            for j in range(d):
                pltpu.async_copy(msgs[p][j], accs[j].at[didxs[p]], ssem,
                                 add=True)
            if count_deg:
                pltpu.async_copy(ones_v, accs[d].at[didxs[p]], ssem, add=True)

        def wait(descs):
            for h in descs:
                h.wait()

        stage(0, 0)
        fire_g(0)

        def pair(i, carry):
            @pl.when(i > 0)
            def _():
                wait(s_descs(1))
            stage(2 * i + 1, 1)
            fire_g(1)
            wait(g_descs(0))
            fire_s(0)
            @pl.when(i < cpw // 2 - 1)
            def _():
                wait(s_descs(0))
                stage(2 * i + 2, 0)
                fire_g(0)
            wait(g_descs(1))
            fire_s(1)
            return carry

        lax.fori_loop(0, cpw // 2, pair, 0)
        wait(s_descs(0))
        wait(s_descs(1))
        plsc.subcore_barrier()

        for a, o in zip(accs, outs):
            pltpu.sync_copy(a.at[pl.ds(s * tr, tr)],
                            o.at[pl.ds(c * n_pad + s * tr, tr)])

    z = jnp.zeros((n_pad,), jnp.float32)
    mesh = plsc.VectorSubcoreMesh(core_axis_name="c", subcore_axis_name="s")
    return pl.kernel(
        body,
        out_type=[jax.ShapeDtypeStruct((NC * n_pad,), jnp.float32)] * nacc,
        mesh=mesh,
        scratch_types=(
            [pltpu.VMEM_SHARED((n_pad,), jnp.float32)] * nacc
            + [pltpu.VMEM((K,), jnp.int32)] * 4
            + [pltpu.VMEM((K,), jnp.float32)] * (2 * d)
            + [pltpu.VMEM((K,), jnp.float32),
               pltpu.SemaphoreType.DMA,
               pltpu.SemaphoreType.DMA]
        ),
        compiler_params=pltpu.CompilerParams(use_tc_tiling_on_sc=False),
    )(*cols, src1, dst1, z)


_BN = 1000


def _mm(a, w, kdim):
    a = a.astype(jnp.bfloat16).astype(jnp.float32)
    w = w.astype(jnp.bfloat16).astype(jnp.float32)
    return sum(a[:, k:k + 1] * w[k:k + 1, :] for k in range(kdim))


def _node_spec(d):
    return pl.BlockSpec((_BN, d), lambda i: (i, 0))


def _w_spec(shape):
    return pl.BlockSpec(shape, lambda i: (0, 0))


def _dense1(ps, pd, x, w1l, b1, w1r):
    n = x.shape[0]

    def body(ps0, ps1, pd0, pd1, x_r, wl_r, b_r, wr_r, h1_r, inv_r):
        inv = 1.0 / jnp.maximum(pd0[...] + pd1[...], 1.0)
        mean = (ps0[...] + ps1[...]) * inv
        h1 = mean * wl_r[...] + b_r[...] + x_r[...] * wr_r[...]
        h1_r[...] = jnp.maximum(h1, 0.0)
        inv_r[...] = inv

    return pl.pallas_call(
        body,
        grid=(n // _BN,),
        in_specs=[_node_spec(1)] * 5 + [_w_spec((1, 4))] * 3,
        out_specs=[_node_spec(4), _node_spec(1)],
        out_shape=[jax.ShapeDtypeStruct((n, 4), jnp.float32),
                   jax.ShapeDtypeStruct((n, 1), jnp.float32)],
    )(ps[0], ps[1], pd[0], pd[1], x, w1l, b1, w1r)


def _dense2(pcols, inv, h1, w2l, b2, w2r):
    n = h1.shape[0]

    def body(p00, p01, p10, p11, p20, p21, p30, p31, inv_r, h1_r,
             wl_r, b_r, wr_r, h2_r):
        iv = inv_r[...]
        mean = jnp.concatenate(
            [(p00[...] + p01[...]) * iv, (p10[...] + p11[...]) * iv,
             (p20[...] + p21[...]) * iv, (p30[...] + p31[...]) * iv], axis=1)
        h2 = _mm(mean, wl_r[...], 4) + b_r[...] + _mm(h1_r[...], wr_r[...], 4)
        h2_r[...] = jnp.maximum(h2, 0.0)

    return pl.pallas_call(
        body,
        grid=(n // _BN,),
        in_specs=([_node_spec(1)] * 9 + [_node_spec(4)]
                  + [_w_spec((4, 4)), _w_spec((1, 4)), _w_spec((4, 4))]),
        out_specs=[_node_spec(4)],
        out_shape=[jax.ShapeDtypeStruct((n, 4), jnp.float32)],
    )(*pcols, inv, h1, w2l, b2, w2r)[0]


def _dense3(pcols, inv, h2, w3l, b3, w3r, wc, bc):
    n = h2.shape[0]

    def body(p00, p01, p10, p11, p20, p21, p30, p31, inv_r, h2_r,
             wl_r, b_r, wr_r, wc_r, bc_r, out_r, h3_r):
        iv = inv_r[...]
        mean = jnp.concatenate(
            [(p00[...] + p01[...]) * iv, (p10[...] + p11[...]) * iv,
             (p20[...] + p21[...]) * iv, (p30[...] + p31[...]) * iv], axis=1)
        h3 = _mm(mean, wl_r[...], 4) + b_r[...] + _mm(h2_r[...], wr_r[...], 4)
        h3 = jnp.maximum(h3, 0.0)
        h3_r[...] = h3
        out_r[...] = _mm(h3, wc_r[...], 2) + bc_r[...]

    return pl.pallas_call(
        body,
        grid=(n // _BN,),
        in_specs=([_node_spec(1)] * 9 + [_node_spec(4)]
                  + [_w_spec((4, 2)), _w_spec((1, 2)), _w_spec((4, 2)),
                     _w_spec((2, 2)), _w_spec((1, 2))]),
        out_specs=[_node_spec(2), _node_spec(2)],
        out_shape=[jax.ShapeDtypeStruct((n, 2), jnp.float32),
                   jax.ShapeDtypeStruct((n, 2), jnp.float32)],
    )(*pcols, inv, h2, w3l, b3, w3r, wc, bc)


def kernel(x, edge_index, w1l, b1, w1r, w2l, b2, w2r, w3l, b3, w3r, wc, bc):
    n = x.shape[0]
    n_pad = ((n + 1 + 8 * NS - 1) // (8 * NS)) * (8 * NS)

    src1, dst1 = _pad_edges(edge_index, n)

    def split(raw):
        return raw[:n].reshape(n, 1), raw[n_pad:n_pad + n].reshape(n, 1)

    outs = _sc_edge_pass((x[:, 0],), src1, dst1, n_pad, True)
    h1, inv = _dense1(split(outs[0]), split(outs[1]), x,
                      w1l, b1.reshape(1, 4), w1r)

    outs = _sc_edge_pass(tuple(h1[:, j] for j in range(4)), src1, dst1,
                         n_pad, False)
    pcols = [p for o in outs for p in split(o)]
    h2 = _dense2(pcols, inv, h1, w2l, b2.reshape(1, 4), w2r)

    outs = _sc_edge_pass(tuple(h2[:, j] for j in range(4)), src1, dst1,
                         n_pad, False)
    pcols = [p for o in outs for p in split(o)]
    out, h3 = _dense3(pcols, inv, h2, w3l, b3.reshape(1, 2), w3r, wc,
                      bc.reshape(1, 2))
    return (out, h3)


# speedup vs baseline: 57.9040x; 1.5882x over previous
import functools

import jax
import jax.numpy as jnp
from jax import lax
from jax.experimental import pallas as pl
from jax.experimental.pallas import tpu as pltpu
from jax.experimental.pallas import tpu_sc as plsc

NC = 2
NS = 16
NW = NC * NS

K = 4000


def _pad_edges(edge_index, n):
    e = edge_index.shape[1]
    unit = K * NW
    e_pad = ((e + unit - 1) // unit) * unit
    pad = e_pad - e
    src = edge_index[0]
    dst = edge_index[1]
    if pad:
        src = jnp.concatenate([src, jnp.zeros((pad,), jnp.int32)])
        dst = jnp.concatenate([dst, jnp.full((pad,), n, jnp.int32)])
    return src, dst


@functools.partial(jax.jit, static_argnums=(3, 4))
def _sc_edge_pass(cols, src1, dst1, n_pad, count_deg):
    d = len(cols)
    nacc = d + (1 if count_deg else 0)
    e_pad = src1.shape[0]
    cpw = e_pad // K // NW
    tr = n_pad // NS
    assert cpw % 2 == 0

    def body(*refs):
        tables = refs[:d]
        src_h, dst_h, z_h = refs[d:d + 3]
        outs = refs[d + 3:d + 3 + 2 * nacc]
        sc = d + 3 + 2 * nacc
        accs = refs[sc:sc + nacc]
        sidxs = refs[sc + nacc:sc + nacc + 2]
        didxs = refs[sc + nacc + 2:sc + nacc + 4]
        msgs = [refs[sc + nacc + 4 + p * d:sc + nacc + 4 + (p + 1) * d]
                for p in range(2)]
        ones_v = refs[-3]
        gsem, ssem = refs[-2], refs[-1]

        c = lax.axis_index("c")
        s = lax.axis_index("s")
        wid = s * NC + c

        if count_deg:
            one = jnp.full((16,), 1.0, jnp.float32)
            def fill(i, carry):
                ones_v[pl.ds(i * 16, 16)] = one
                return carry
            lax.fori_loop(0, K // 16, fill, 0)

        for a in accs:
            pltpu.sync_copy(z_h.at[pl.ds(s * tr, tr)], a.at[pl.ds(s * tr, tr)])
        plsc.subcore_barrier()

        def stage(o, p):
            base = (wid * cpw + o) * K
            pltpu.sync_copy(src_h.at[pl.ds(base, K)], sidxs[p])
            pltpu.sync_copy(dst_h.at[pl.ds(base, K)], didxs[p])

        def g_descs(p):
            return [pltpu.make_async_copy(tables[j].at[sidxs[p]],
                                          msgs[p][j], gsem)
                    for j in range(d)]

        def s_descs(p):
            hs = [pltpu.make_async_copy(msgs[p][j], accs[j].at[didxs[p]],
                                        ssem)
                  for j in range(d)]
            if count_deg:
                hs.append(pltpu.make_async_copy(ones_v, accs[d].at[didxs[p]],
                                                ssem))
            return hs

        def fire_g(p):
            for j in range(d):
                pltpu.async_copy(tables[j].at[sidxs[p]], msgs[p][j], gsem)

        def fire_s(p):
            for j in range(d):
                pltpu.async_copy(msgs[p][j], accs[j].at[didxs[p]], ssem,
                                 add=True)
            if count_deg:
                pltpu.async_copy(ones_v, accs[d].at[didxs[p]], ssem, add=True)

        def wait(descs):
            for h in descs:
                h.wait()

        stage(0, 0)
        fire_g(0)

        def pair(i, carry):
            @pl.when(i > 0)
            def _():
                wait(s_descs(1))
            stage(2 * i + 1, 1)
            fire_g(1)
            wait(g_descs(0))
            fire_s(0)
            @pl.when(i < cpw // 2 - 1)
            def _():
                wait(s_descs(0))
                stage(2 * i + 2, 0)
                fire_g(0)
            wait(g_descs(1))
            fire_s(1)
            return carry

        lax.fori_loop(0, cpw // 2, pair, 0)
        wait(s_descs(0))
        wait(s_descs(1))
        plsc.subcore_barrier()

        for j, a in enumerate(accs):
            def wr(aa, jj):
                @pl.when(c == 0)
                def _():
                    pltpu.sync_copy(aa.at[pl.ds(s * tr, tr)],
                                    outs[2 * jj].at[pl.ds(s * tr, tr)])
                @pl.when(c == 1)
                def _():
                    pltpu.sync_copy(aa.at[pl.ds(s * tr, tr)],
                                    outs[2 * jj + 1].at[pl.ds(s * tr, tr)])
            wr(a, j)

    z = jnp.zeros((n_pad,), jnp.float32)
    mesh = plsc.VectorSubcoreMesh(core_axis_name="c", subcore_axis_name="s")
    return pl.kernel(
        body,
        out_type=[jax.ShapeDtypeStruct((n_pad,), jnp.float32)] * (2 * nacc),
        mesh=mesh,
        scratch_types=(
            [pltpu.VMEM_SHARED((n_pad,), jnp.float32)] * nacc
            + [pltpu.VMEM((K,), jnp.int32)] * 4
            + [pltpu.VMEM((K,), jnp.float32)] * (2 * d)
            + [pltpu.VMEM((K,), jnp.float32),
               pltpu.SemaphoreType.DMA,
               pltpu.SemaphoreType.DMA]
        ),
        compiler_params=pltpu.CompilerParams(use_tc_tiling_on_sc=False),
    )(*cols, src1, dst1, z)


def _r16(a):
    return a.astype(jnp.bfloat16).astype(jnp.float32)


def _dense_call(body, n_in, n_smem, n_out, shape):
    return pl.pallas_call(
        body,
        in_specs=[pl.BlockSpec(shape, lambda: (0, 0))] * n_in
        + [pl.BlockSpec(memory_space=pltpu.SMEM)] * n_smem,
        out_specs=[pl.BlockSpec(shape, lambda: (0, 0))] * n_out,
        out_shape=[jax.ShapeDtypeStruct(shape, jnp.float32)] * n_out,
    )


def _dense1(ps, pd, xc, w1l, b1, w1r, shape):
    def body(ps0, ps1, pd0, pd1, x_r, wl, b, wr, h0, h1, h2, h3, inv_r):
        inv = 1.0 / jnp.maximum(pd0[...] + pd1[...], 1.0)
        mean = (ps0[...] + ps1[...]) * inv
        xv = x_r[...]
        outs = [h0, h1, h2, h3]
        for f in range(4):
            h = mean * wl[0, f] + b[f] + xv * wr[0, f]
            outs[f][...] = jnp.maximum(h, 0.0)
        inv_r[...] = inv

    return _dense_call(body, 5, 3, 5, shape)(
        ps[0], ps[1], pd[0], pd[1], xc, w1l, b1, w1r)


def _dense2(p, inv, h1c, w2l, b2, w2r, shape):
    def body(p00, p01, p10, p11, p20, p21, p30, p31, inv_r,
             a0, a1, a2, a3, wl, b, wr, o0, o1, o2, o3):
        iv = inv_r[...]
        ps = [(p00, p01), (p10, p11), (p20, p21), (p30, p31)]
        mean = [_r16((u[...] + v[...]) * iv) for u, v in ps]
        hv = [_r16(a[...]) for a in (a0, a1, a2, a3)]
        outs = [o0, o1, o2, o3]
        for f in range(4):
            acc = b[f]
            for k in range(4):
                acc = acc + mean[k] * wl[k, f] + hv[k] * wr[k, f]
            outs[f][...] = jnp.maximum(acc, 0.0)

    return _dense_call(body, 13, 3, 4, shape)(*p, inv, *h1c, w2l, b2, w2r)


def _dense3(p, inv, h2c, w3l, b3, w3r, wc, bc, shape):
    def body(p00, p01, p10, p11, p20, p21, p30, p31, inv_r,
             a0, a1, a2, a3, wl, b, wr, wcc, bcc, o0, o1, g0, g1):
        iv = inv_r[...]
        ps = [(p00, p01), (p10, p11), (p20, p21), (p30, p31)]
        mean = [_r16((u[...] + v[...]) * iv) for u, v in ps]
        hv = [_r16(a[...]) for a in (a0, a1, a2, a3)]
        h3 = []
        for f in range(2):
            acc = b[f]
            for k in range(4):
                acc = acc + mean[k] * wl[k, f] + hv[k] * wr[k, f]
            h3.append(jnp.maximum(acc, 0.0))
        g0[...] = h3[0]
        g1[...] = h3[1]
        h316 = [_r16(h) for h in h3]
        for f, o in enumerate((o0, o1)):
            o[...] = h316[0] * wcc[0, f] + h316[1] * wcc[1, f] + bcc[f]

    return _dense_call(body, 13, 5, 4, shape)(
        *p, inv, *h2c, w3l, b3, w3r, wc, bc)


def kernel(x, edge_index, w1l, b1, w1r, w2l, b2, w2r, w3l, b3, w3r, wc, bc):
    n = x.shape[0]
    n_pad = ((n + 1 + 127) // 128) * 128
    shape = (n_pad // 128, 128)

    src1, dst1 = _pad_edges(edge_index, n)

    w2l16, w2r16 = _r16(w2l), _r16(w2r)
    w3l16, w3r16, wc16 = _r16(w3l), _r16(w3r), _r16(wc)

    def v2(a):
        return a.reshape(shape)

    xc = jnp.pad(x[:, 0], (0, n_pad - n))
    o = _sc_edge_pass((xc,), src1, dst1, n_pad, True)
    h1c0, h1c1, h1c2, h1c3, inv = _dense1(
        (v2(o[0]), v2(o[1])), (v2(o[2]), v2(o[3])), v2(xc),
        w1l, b1, w1r, shape)
    h1c = [h1c0, h1c1, h1c2, h1c3]

    o = _sc_edge_pass(tuple(a.reshape(-1) for a in h1c), src1, dst1,
                      n_pad, False)
    h2c = _dense2([v2(a) for a in o], inv, h1c, w2l16, b2, w2r16, shape)

    o = _sc_edge_pass(tuple(a.reshape(-1) for a in h2c), src1, dst1,
                      n_pad, False)
    o0, o1, g0, g1 = _dense3([v2(a) for a in o], inv, h2c,
                             w3l16, b3, w3r16, wc16, bc, shape)

    out = jnp.stack([o0.reshape(-1)[:n], o1.reshape(-1)[:n]], axis=1)
    h3 = jnp.stack([g0.reshape(-1)[:n], g1.reshape(-1)[:n]], axis=1)
    return (out, h3)
